# baseline (speedup 1.0000x reference)
import functools

import jax
import jax.numpy as jnp
import numpy as np
from jax import lax
from jax.experimental import pallas as pl
from jax.experimental.pallas import tpu as pltpu
from jax.experimental.pallas import tpu_sc as plsc

N_NODES = 100000
N_EDGES = 1600000
HIDDEN = 50
NCHUNK = 4
CW = 16
CHUNK_COLS = ((0, 13), (13, 13), (26, 13), (39, 11))

N_SUB = 16
BLK_E = 400
EDGES_PER_SUB = N_EDGES // N_SUB
N_BLK = EDGES_PER_SUB // BLK_E
NP = 100096
ROWS_PER_SUB = NP // N_SUB
ZROWS = 136

NODE_BLK = 4352
EDGE_BLK = 4000

_SEL = np.zeros((NCHUNK, HIDDEN, CW), np.float32)
for _k, (_c0, _w) in enumerate(CHUNK_COLS):
    _SEL[_k, _c0 + np.arange(_w), np.arange(_w)] = 1.0

_g_of_c = np.arange(HIDDEN) // 5
_GM = np.zeros((HIDDEN, 10), np.float32)
_GM[np.arange(HIDDEN), _g_of_c] = 0.2
_PM = (_g_of_c[None, :] == np.arange(10)[:, None]).astype(np.float32)


def _split_chunks(x, blk, rows_out):
    n = x.shape[0]
    nblk = rows_out // blk

    def body(x_ref, s_ref, o_ref):
        o_ref[...] = jnp.dot(x_ref[...], s_ref[0],
                             preferred_element_type=jnp.float32)

    return pl.pallas_call(
        body,
        grid=(nblk, NCHUNK),
        in_specs=[
            pl.BlockSpec((blk, HIDDEN), lambda i, k: (i, 0)),
            pl.BlockSpec((1, HIDDEN, CW), lambda i, k: (k, 0, 0)),
        ],
        out_specs=pl.BlockSpec((blk, CW), lambda i, k: (k * nblk + i, 0)),
        out_shape=jax.ShapeDtypeStruct((NCHUNK * rows_out, CW), jnp.float32),
    )(x, jnp.asarray(_SEL))


def _sc_agg(hc_flat, attr_flat, src, dst):
    mesh = plsc.VectorSubcoreMesh(core_axis_name="c", subcore_axis_name="s")

    @functools.partial(
        pl.kernel,
        out_type=jax.ShapeDtypeStruct((NCHUNK * NP, CW), jnp.float32),
        mesh=mesh,
        scratch_types=[
            pltpu.VMEM_SHARED((NP, CW), jnp.float32),
            pltpu.VMEM((2, BLK_E), jnp.int32),
            pltpu.VMEM((2, BLK_E), jnp.int32),
            pltpu.VMEM((2, BLK_E, CW), jnp.float32),
            pltpu.VMEM((2, BLK_E, CW), jnp.float32),
            pltpu.VMEM((ZROWS, CW), jnp.float32),
            pltpu.SemaphoreType.DMA((2,)),
            pltpu.SemaphoreType.DMA((2,)),
            pltpu.SemaphoreType.DMA((2,)),
            pltpu.SemaphoreType.DMA((2,)),
            pltpu.SemaphoreType.DMA((2,)),
        ],
        compiler_params=pltpu.CompilerParams(use_tc_tiling_on_sc=False),
    )
    def k(hc_hbm, attr_hbm, src_hbm, dst_hbm, out_hbm,
          acc_sh, src_v, dst_v, hrow_v, attr_v, zero_v,
          s_src, s_dst, s_att, s_gat, s_sca):
        c = lax.axis_index("c")
        s = lax.axis_index("s")

        zvec = jnp.zeros((CW,), jnp.float32)

        @pl.loop(0, ZROWS, unroll=8)
        def _(i):
            zero_v[i] = zvec

        def issue_in(jb, u, kk):
            e0 = s * EDGES_PER_SUB + jb * BLK_E
            pltpu.async_copy(src_hbm.at[pl.ds(e0, BLK_E)], src_v.at[u],
                             s_src.at[u])
            pltpu.async_copy(dst_hbm.at[pl.ds(e0, BLK_E)], dst_v.at[u],
                             s_dst.at[u])
            pltpu.async_copy(attr_hbm.at[pl.ds(kk * N_EDGES + e0, BLK_E)],
                             attr_v.at[u], s_att.at[u])

        def wait_in(u):
            pltpu.make_async_copy(src_hbm.at[pl.ds(0, BLK_E)], src_v.at[u],
                                  s_src.at[u]).wait()
            pltpu.make_async_copy(dst_hbm.at[pl.ds(0, BLK_E)], dst_v.at[u],
                                  s_dst.at[u]).wait()
            pltpu.make_async_copy(attr_hbm.at[pl.ds(0, BLK_E)], attr_v.at[u],
                                  s_att.at[u]).wait()

        def wait_scatter(u):
            pltpu.make_async_copy(hrow_v.at[u], acc_sh.at[dst_v.at[u]],
                                  s_sca.at[u]).wait()

        for chunk_i in range(NCHUNK // 2):
            kk = 2 * c + chunk_i

            @pl.loop(0, ROWS_PER_SUB, step=ZROWS)
            def _(r):
                pltpu.sync_copy(zero_v,
                                acc_sh.at[pl.ds(s * ROWS_PER_SUB + r, ZROWS)])

            plsc.subcore_barrier()

            off = kk * NP
            issue_in(0, 0, kk)

            @pl.loop(0, N_BLK, step=2)
            def _(j):
                for u in range(2):
                    jb = j + u
                    un = 1 - u
                    @pl.when(jb + 1 < N_BLK)
                    def _():
                        @pl.when(jb >= 1)
                        def _():
                            wait_scatter(un)

                        issue_in(jb + 1, un, kk)

                    wait_in(u)

                    @pl.loop(0, BLK_E, step=16, unroll=8)
                    def _(i):
                        src_v[u, pl.ds(i, 16)] = src_v[u, pl.ds(i, 16)] + off

                    pltpu.async_copy(hc_hbm.at[src_v.at[u]], hrow_v.at[u],
                                     s_gat.at[u]).wait()

                    @pl.loop(0, BLK_E, unroll=16)
                    def _(r):
                        hrow_v[u, r] = jnp.maximum(
                            hrow_v[u, r] + attr_v[u, r], 0.0)

                    pltpu.async_copy(hrow_v.at[u], acc_sh.at[dst_v.at[u]],
                                     s_sca.at[u], add=True)

            wait_scatter(0)
            wait_scatter(1)
            plsc.subcore_barrier()

            pltpu.sync_copy(
                acc_sh.at[pl.ds(s * ROWS_PER_SUB, ROWS_PER_SUB)],
                out_hbm.at[pl.ds(kk * NP + s * ROWS_PER_SUB, ROWS_PER_SUB)])

    return k(hc_flat, attr_flat, src, dst)


def _mlp_gn(h, agg_flat, w1t, b1, w2t, b2, epsl, gamma, beta, last):
    blk = NODE_BLK
    nblk = NP // blk
    gm = jnp.asarray(_GM)
    pm = jnp.asarray(_PM)

    def body(h_ref, a0_ref, a1_ref, a2_ref, a3_ref, w1_ref, b1_ref, w2_ref,
             b2_ref, gm_ref, pm_ref, gamma_ref, beta_ref, eps_ref, o_ref):
        h_blk = h_ref[...]
        a_refs = (a0_ref, a1_ref, a2_ref, a3_ref)
        agg_blk = jnp.concatenate(
            [a_refs[ci][:, :w] for ci, (c0, w) in enumerate(CHUNK_COLS)],
            axis=1)
        z = (1.0 + eps_ref[0, 0]) * h_blk + agg_blk
        z = jnp.maximum(
            jnp.dot(z, w1_ref[...], preferred_element_type=jnp.float32)
            + b1_ref[...], 0.0)
        z = jnp.dot(z, w2_ref[...], preferred_element_type=jnp.float32) \
            + b2_ref[...]
        mean = jnp.dot(z, gm_ref[...], preferred_element_type=jnp.float32)
        zc = z - jnp.dot(mean, pm_ref[...], preferred_element_type=jnp.float32)
        var = jnp.dot(zc * zc, gm_ref[...], preferred_element_type=jnp.float32)
        rstd = lax.rsqrt(var + 1e-5)
        zn = zc * jnp.dot(rstd, pm_ref[...], preferred_element_type=jnp.float32)
        out = zn * gamma_ref[...] + beta_ref[...]
        if not last:
            out = jnp.maximum(out, 0.0)
        o_ref[...] = out

    def chunk_spec(kc):
        return pl.BlockSpec((blk, CW), lambda i, kc=kc: (kc * nblk + i, 0))

    full = lambda shape: pl.BlockSpec(shape, lambda i: tuple(0 for _ in shape))
    in_specs = [
        pl.BlockSpec((blk, HIDDEN), lambda i: (i, 0)),
        chunk_spec(0), chunk_spec(1), chunk_spec(2), chunk_spec(3),
        full((HIDDEN, HIDDEN)),
        full((1, HIDDEN)),
        full((HIDDEN, HIDDEN)),
        full((1, HIDDEN)),
        full((HIDDEN, 10)),
        full((10, HIDDEN)),
        full((1, HIDDEN)),
        full((1, HIDDEN)),
        full((1, 1)),
    ]
    return pl.pallas_call(
        body,
        grid=(nblk,),
        in_specs=in_specs,
        out_specs=pl.BlockSpec((blk, HIDDEN), lambda i: (i, 0)),
        out_shape=jax.ShapeDtypeStruct((N_NODES, HIDDEN), jnp.float32),
    )(h, agg_flat, agg_flat, agg_flat, agg_flat, w1t, b1.reshape(1, HIDDEN),
      w2t, b2.reshape(1, HIDDEN), gm, pm, gamma.reshape(1, HIDDEN),
      beta.reshape(1, HIDDEN), epsl.reshape(1, 1))


def kernel(x, edge_index, edge_attr, W1, b1, W2, b2, eps, gamma, beta):
    src = edge_index[0]
    dst = edge_index[1]
    ac = _split_chunks(edge_attr, EDGE_BLK, N_EDGES)
    hc = _split_chunks(x, NODE_BLK, NP)
    h = x
    for l in range(2):
        agg = _sc_agg(hc, ac, src, dst)
        h = _mlp_gn(h, agg, W1[l].T, b1[l], W2[l].T, b2[l], eps[l],
                    gamma[l], beta[l], last=(l == 1))
        if l == 0:
            hc = _split_chunks(h, NODE_BLK, NP)
    return h


# speedup vs baseline: 1.2768x; 1.2768x over previous
import functools

import jax
import jax.numpy as jnp
import numpy as np
from jax import lax
from jax.experimental import pallas as pl
from jax.experimental.pallas import tpu as pltpu
from jax.experimental.pallas import tpu_sc as plsc

N_NODES = 100000
N_EDGES = 1600000
HIDDEN = 50
NCHUNK = 4
CW = 16
CHUNK_COLS = ((0, 13), (13, 13), (26, 13), (39, 11))

N_SUB = 16
BLK_E = 400
EDGES_PER_SUB = N_EDGES // N_SUB
N_BLK = EDGES_PER_SUB // BLK_E
NP = 100096
ROWS_PER_SUB = NP // N_SUB
ZROWS = 136

NODE_BLK = 4352
EDGE_BLK = 3200

_SEL = np.zeros((NCHUNK, HIDDEN, CW), np.float32)
for _k, (_c0, _w) in enumerate(CHUNK_COLS):
    _SEL[_k, _c0 + np.arange(_w), np.arange(_w)] = 1.0

_g_of_c = np.arange(HIDDEN) // 5
_GM = np.zeros((HIDDEN, 10), np.float32)
_GM[np.arange(HIDDEN), _g_of_c] = 0.2
_PM = (_g_of_c[None, :] == np.arange(10)[:, None]).astype(np.float32)


def _split_chunks(x, blk, rows_out):
    n = x.shape[0]
    nblk = rows_out // blk

    def body(x_ref, s_ref, o_ref):
        o_ref[...] = jnp.dot(x_ref[...], s_ref[0],
                             preferred_element_type=jnp.float32)

    return pl.pallas_call(
        body,
        grid=(nblk, NCHUNK),
        in_specs=[
            pl.BlockSpec((blk, HIDDEN), lambda i, k: (i, 0)),
            pl.BlockSpec((1, HIDDEN, CW), lambda i, k: (k, 0, 0)),
        ],
        out_specs=pl.BlockSpec((blk, CW), lambda i, k: (k * nblk + i, 0)),
        out_shape=jax.ShapeDtypeStruct((NCHUNK * rows_out, CW), jnp.float32),
    )(x, jnp.asarray(_SEL))


def _split_chunks_wide(x, blk, rows_out):
    n = x.shape[0]
    nblk = rows_out // blk
    wblk = blk * CW // 128

    def body(x_ref, s_ref, o_ref):
        for p in range(8):
            o_ref[:, CW * p:CW * (p + 1)] = jnp.dot(
                x_ref[:, p, :], s_ref[0], preferred_element_type=jnp.float32)

    return pl.pallas_call(
        body,
        grid=(nblk, NCHUNK),
        in_specs=[
            pl.BlockSpec((wblk, 8, HIDDEN), lambda i, k: (i, 0, 0)),
            pl.BlockSpec((1, HIDDEN, CW), lambda i, k: (k, 0, 0)),
        ],
        out_specs=pl.BlockSpec((wblk, 128), lambda i, k: (k * nblk + i, 0)),
        out_shape=jax.ShapeDtypeStruct((NCHUNK * rows_out * CW // 128, 128),
                                       jnp.float32),
    )(x.reshape(n // 8, 8, HIDDEN), jnp.asarray(_SEL))


def _sc_agg(hc_flat, attr_flat, src, dst):
    mesh = plsc.VectorSubcoreMesh(core_axis_name="c", subcore_axis_name="s")

    @functools.partial(
        pl.kernel,
        out_type=jax.ShapeDtypeStruct((NCHUNK * NP, CW), jnp.float32),
        mesh=mesh,
        scratch_types=[
            pltpu.VMEM_SHARED((NP, CW), jnp.float32),
            pltpu.VMEM((2, BLK_E), jnp.int32),
            pltpu.VMEM((2, BLK_E), jnp.int32),
            pltpu.VMEM((2, BLK_E, CW), jnp.float32),
            pltpu.VMEM((2, BLK_E * CW // 128, 128), jnp.float32),
            pltpu.VMEM((ZROWS, CW), jnp.float32),
            pltpu.SemaphoreType.DMA((2,)),
            pltpu.SemaphoreType.DMA((2,)),
            pltpu.SemaphoreType.DMA((2,)),
            pltpu.SemaphoreType.DMA((2,)),
            pltpu.SemaphoreType.DMA((2,)),
        ],
        compiler_params=pltpu.CompilerParams(use_tc_tiling_on_sc=False),
    )
    def k(hc_hbm, attr_hbm, src_hbm, dst_hbm, out_hbm,
          acc_sh, src_v, dst_v, hrow_v, attr_v, zero_v,
          s_src, s_dst, s_att, s_gat, s_sca):
        c = lax.axis_index("c")
        s = lax.axis_index("s")

        zvec = jnp.zeros((CW,), jnp.float32)

        @pl.loop(0, ZROWS, unroll=8)
        def _(i):
            zero_v[i] = zvec

        wblk = BLK_E * CW // 128

        def issue_in(jb, u, kk):
            e0 = s * EDGES_PER_SUB + jb * BLK_E
            w0 = (kk * N_EDGES + e0) * CW // 128
            pltpu.async_copy(src_hbm.at[pl.ds(e0, BLK_E)], src_v.at[u],
                             s_src.at[u])
            pltpu.async_copy(dst_hbm.at[pl.ds(e0, BLK_E)], dst_v.at[u],
                             s_dst.at[u])
            pltpu.async_copy(attr_hbm.at[pl.ds(w0, wblk)],
                             attr_v.at[u], s_att.at[u])

        def wait_in(u):
            pltpu.make_async_copy(src_hbm.at[pl.ds(0, BLK_E)], src_v.at[u],
                                  s_src.at[u]).wait()
            pltpu.make_async_copy(dst_hbm.at[pl.ds(0, BLK_E)], dst_v.at[u],
                                  s_dst.at[u]).wait()
            pltpu.make_async_copy(attr_hbm.at[pl.ds(0, wblk)], attr_v.at[u],
                                  s_att.at[u]).wait()

        def wait_scatter(u):
            pltpu.make_async_copy(hrow_v.at[u], acc_sh.at[dst_v.at[u]],
                                  s_sca.at[u]).wait()

        for chunk_i in range(NCHUNK // 2):
            kk = 2 * c + chunk_i

            @pl.loop(0, ROWS_PER_SUB, step=ZROWS)
            def _(r):
                pltpu.sync_copy(zero_v,
                                acc_sh.at[pl.ds(s * ROWS_PER_SUB + r, ZROWS)])

            plsc.subcore_barrier()

            off = kk * NP
            issue_in(0, 0, kk)

            @pl.loop(0, N_BLK, step=2)
            def _(j):
                for u in range(2):
                    jb = j + u
                    un = 1 - u
                    @pl.when(jb + 1 < N_BLK)
                    def _():
                        @pl.when(jb >= 1)
                        def _():
                            wait_scatter(un)

                        issue_in(jb + 1, un, kk)

                    wait_in(u)

                    @pl.loop(0, BLK_E, step=16, unroll=8)
                    def _(i):
                        src_v[u, pl.ds(i, 16)] = src_v[u, pl.ds(i, 16)] + off

                    pltpu.async_copy(hc_hbm.at[src_v.at[u]], hrow_v.at[u],
                                     s_gat.at[u]).wait()

                    @pl.loop(0, wblk, unroll=2)
                    def _(q):
                        for r in range(8):
                            hrow_v[u, 8 * q + r] = jnp.maximum(
                                hrow_v[u, 8 * q + r]
                                + attr_v[u, q, 16 * r:16 * (r + 1)], 0.0)

                    pltpu.async_copy(hrow_v.at[u], acc_sh.at[dst_v.at[u]],
                                     s_sca.at[u], add=True)

            wait_scatter(0)
            wait_scatter(1)
            plsc.subcore_barrier()

            pltpu.sync_copy(
                acc_sh.at[pl.ds(s * ROWS_PER_SUB, ROWS_PER_SUB)],
                out_hbm.at[pl.ds(kk * NP + s * ROWS_PER_SUB, ROWS_PER_SUB)])

    return k(hc_flat, attr_flat, src, dst)


def _mlp_gn(h, agg_flat, w1t, b1, w2t, b2, epsl, gamma, beta, last):
    blk = NODE_BLK
    nblk = NP // blk
    gm = jnp.asarray(_GM)
    pm = jnp.asarray(_PM)

    def body(h_ref, a0_ref, a1_ref, a2_ref, a3_ref, w1_ref, b1_ref, w2_ref,
             b2_ref, gm_ref, pm_ref, gamma_ref, beta_ref, eps_ref, o_ref):
        h_blk = h_ref[...]
        a_refs = (a0_ref, a1_ref, a2_ref, a3_ref)
        agg_blk = jnp.concatenate(
            [a_refs[ci][:, :w] for ci, (c0, w) in enumerate(CHUNK_COLS)],
            axis=1)
        z = (1.0 + eps_ref[0, 0]) * h_blk + agg_blk
        z = jnp.maximum(
            jnp.dot(z, w1_ref[...], preferred_element_type=jnp.float32)
            + b1_ref[...], 0.0)
        z = jnp.dot(z, w2_ref[...], preferred_element_type=jnp.float32) \
            + b2_ref[...]
        mean = jnp.dot(z, gm_ref[...], preferred_element_type=jnp.float32)
        zc = z - jnp.dot(mean, pm_ref[...], preferred_element_type=jnp.float32)
        var = jnp.dot(zc * zc, gm_ref[...], preferred_element_type=jnp.float32)
        rstd = lax.rsqrt(var + 1e-5)
        zn = zc * jnp.dot(rstd, pm_ref[...], preferred_element_type=jnp.float32)
        out = zn * gamma_ref[...] + beta_ref[...]
        if not last:
            out = jnp.maximum(out, 0.0)
        o_ref[...] = out

    def chunk_spec(kc):
        return pl.BlockSpec((blk, CW), lambda i, kc=kc: (kc * nblk + i, 0))

    full = lambda shape: pl.BlockSpec(shape, lambda i: tuple(0 for _ in shape))
    in_specs = [
        pl.BlockSpec((blk, HIDDEN), lambda i: (i, 0)),
        chunk_spec(0), chunk_spec(1), chunk_spec(2), chunk_spec(3),
        full((HIDDEN, HIDDEN)),
        full((1, HIDDEN)),
        full((HIDDEN, HIDDEN)),
        full((1, HIDDEN)),
        full((HIDDEN, 10)),
        full((10, HIDDEN)),
        full((1, HIDDEN)),
        full((1, HIDDEN)),
        full((1, 1)),
    ]
    return pl.pallas_call(
        body,
        grid=(nblk,),
        in_specs=in_specs,
        out_specs=pl.BlockSpec((blk, HIDDEN), lambda i: (i, 0)),
        out_shape=jax.ShapeDtypeStruct((N_NODES, HIDDEN), jnp.float32),
    )(h, agg_flat, agg_flat, agg_flat, agg_flat, w1t, b1.reshape(1, HIDDEN),
      w2t, b2.reshape(1, HIDDEN), gm, pm, gamma.reshape(1, HIDDEN),
      beta.reshape(1, HIDDEN), epsl.reshape(1, 1))


def kernel(x, edge_index, edge_attr, W1, b1, W2, b2, eps, gamma, beta):
    src = edge_index[0]
    dst = edge_index[1]
    ac = _split_chunks_wide(edge_attr, EDGE_BLK, N_EDGES)
    hc = _split_chunks(x, NODE_BLK, NP)
    h = x
    for l in range(2):
        agg = _sc_agg(hc, ac, src, dst)
        h = _mlp_gn(h, agg, W1[l].T, b1[l], W2[l].T, b2[l], eps[l],
                    gamma[l], beta[l], last=(l == 1))
        if l == 0:
            hc = _split_chunks(h, NODE_BLK, NP)
    return h


# speedup vs baseline: 1.3799x; 1.0808x over previous
import functools

import jax
import jax.numpy as jnp
import numpy as np
from jax import lax
from jax.experimental import pallas as pl
from jax.experimental.pallas import tpu as pltpu
from jax.experimental.pallas import tpu_sc as plsc

N_NODES = 100000
N_EDGES = 1600000
HIDDEN = 50
NCHUNK = 4
CW = 16
CHUNK_COLS = ((0, 13), (13, 13), (26, 13), (39, 11))

N_SUB = 16
BLK_E = 400
EDGES_PER_SUB = N_EDGES // N_SUB
N_BLK = EDGES_PER_SUB // BLK_E
NP = 100096
ROWS_PER_SUB = NP // N_SUB
ZROWS = 136

NODE_BLK = 4352
EDGE_BLK = 12800

_SEL = np.zeros((NCHUNK, HIDDEN, CW), np.float32)
for _k, (_c0, _w) in enumerate(CHUNK_COLS):
    _SEL[_k, _c0 + np.arange(_w), np.arange(_w)] = 1.0

_g_of_c = np.arange(HIDDEN) // 5
_GM = np.zeros((HIDDEN, 10), np.float32)
_GM[np.arange(HIDDEN), _g_of_c] = 0.2
_PM = (_g_of_c[None, :] == np.arange(10)[:, None]).astype(np.float32)


def _split_chunks(x, blk, rows_out):
    n = x.shape[0]
    nblk = rows_out // blk

    def body(x_ref, s_ref, o_ref):
        o_ref[...] = jnp.dot(x_ref[...], s_ref[0],
                             preferred_element_type=jnp.float32)

    return pl.pallas_call(
        body,
        grid=(nblk, NCHUNK),
        in_specs=[
            pl.BlockSpec((blk, HIDDEN), lambda i, k: (i, 0)),
            pl.BlockSpec((1, HIDDEN, CW), lambda i, k: (k, 0, 0)),
        ],
        out_specs=pl.BlockSpec((blk, CW), lambda i, k: (k * nblk + i, 0)),
        out_shape=jax.ShapeDtypeStruct((NCHUNK * rows_out, CW), jnp.float32),
    )(x, jnp.asarray(_SEL))


def _split_chunks_wide(x, blk, rows_out):
    n = x.shape[0]
    nblk = rows_out // blk
    wblk = blk * CW // 128

    def body(x_ref, s_ref, o_ref):
        for p in range(8):
            o_ref[:, CW * p:CW * (p + 1)] = jnp.dot(
                x_ref[:, p, :], s_ref[0], preferred_element_type=jnp.float32)

    return pl.pallas_call(
        body,
        grid=(nblk, NCHUNK),
        in_specs=[
            pl.BlockSpec((wblk, 8, HIDDEN), lambda i, k: (i, 0, 0)),
            pl.BlockSpec((1, HIDDEN, CW), lambda i, k: (k, 0, 0)),
        ],
        out_specs=pl.BlockSpec((wblk, 128), lambda i, k: (k * nblk + i, 0)),
        out_shape=jax.ShapeDtypeStruct((NCHUNK * rows_out * CW // 128, 128),
                                       jnp.float32),
    )(x.reshape(n // 8, 8, HIDDEN), jnp.asarray(_SEL))


def _sc_agg(hc_flat, attr_flat, src, dst):
    mesh = plsc.VectorSubcoreMesh(core_axis_name="c", subcore_axis_name="s")

    @functools.partial(
        pl.kernel,
        out_type=jax.ShapeDtypeStruct((NCHUNK * NP, CW), jnp.float32),
        mesh=mesh,
        scratch_types=[
            pltpu.VMEM_SHARED((NP, CW), jnp.float32),
            pltpu.VMEM((2, BLK_E), jnp.int32),
            pltpu.VMEM((2, BLK_E), jnp.int32),
            pltpu.VMEM((2, BLK_E, CW), jnp.float32),
            pltpu.VMEM((2, BLK_E * CW // 128, 128), jnp.float32),
            pltpu.VMEM((ZROWS, CW), jnp.float32),
            pltpu.SemaphoreType.DMA((2,)),
            pltpu.SemaphoreType.DMA((2,)),
            pltpu.SemaphoreType.DMA((2,)),
            pltpu.SemaphoreType.DMA((2,)),
            pltpu.SemaphoreType.DMA((2,)),
        ],
        compiler_params=pltpu.CompilerParams(use_tc_tiling_on_sc=False),
    )
    def k(hc_hbm, attr_hbm, src_hbm, dst_hbm, out_hbm,
          acc_sh, src_v, dst_v, hrow_v, attr_v, zero_v,
          s_src, s_dst, s_att, s_gat, s_sca):
        c = lax.axis_index("c")
        s = lax.axis_index("s")

        zvec = jnp.zeros((CW,), jnp.float32)

        @pl.loop(0, ZROWS, unroll=8)
        def _(i):
            zero_v[i] = zvec

        wblk = BLK_E * CW // 128

        def issue_in(jb, u, kk):
            e0 = s * EDGES_PER_SUB + jb * BLK_E
            w0 = (kk * N_EDGES + e0) * CW // 128
            pltpu.async_copy(src_hbm.at[pl.ds(e0, BLK_E)], src_v.at[u],
                             s_src.at[u])
            pltpu.async_copy(dst_hbm.at[pl.ds(e0, BLK_E)], dst_v.at[u],
                             s_dst.at[u])
            pltpu.async_copy(attr_hbm.at[pl.ds(w0, wblk)],
                             attr_v.at[u], s_att.at[u])

        def wait_in(u):
            pltpu.make_async_copy(src_hbm.at[pl.ds(0, BLK_E)], src_v.at[u],
                                  s_src.at[u]).wait()
            pltpu.make_async_copy(dst_hbm.at[pl.ds(0, BLK_E)], dst_v.at[u],
                                  s_dst.at[u]).wait()
            pltpu.make_async_copy(attr_hbm.at[pl.ds(0, wblk)], attr_v.at[u],
                                  s_att.at[u]).wait()

        def wait_scatter(u):
            pltpu.make_async_copy(hrow_v.at[u], acc_sh.at[dst_v.at[u]],
                                  s_sca.at[u]).wait()

        for chunk_i in range(NCHUNK // 2):
            kk = 2 * c + chunk_i

            @pl.loop(0, ROWS_PER_SUB, step=ZROWS)
            def _(r):
                pltpu.sync_copy(zero_v,
                                acc_sh.at[pl.ds(s * ROWS_PER_SUB + r, ZROWS)])

            plsc.subcore_barrier()

            off = kk * NP
            issue_in(0, 0, kk)

            @pl.loop(0, N_BLK, step=2)
            def _(j):
                for u in range(2):
                    jb = j + u
                    un = 1 - u
                    @pl.when(jb + 1 < N_BLK)
                    def _():
                        @pl.when(jb >= 1)
                        def _():
                            wait_scatter(un)

                        issue_in(jb + 1, un, kk)

                    wait_in(u)

                    @pl.loop(0, BLK_E, step=16, unroll=8)
                    def _(i):
                        src_v[u, pl.ds(i, 16)] = src_v[u, pl.ds(i, 16)] + off

                    pltpu.async_copy(hc_hbm.at[src_v.at[u]], hrow_v.at[u],
                                     s_gat.at[u]).wait()

                    @pl.loop(0, wblk, unroll=2)
                    def _(q):
                        for r in range(8):
                            hrow_v[u, 8 * q + r] = jnp.maximum(
                                hrow_v[u, 8 * q + r]
                                + attr_v[u, q, 16 * r:16 * (r + 1)], 0.0)

                    pltpu.async_copy(hrow_v.at[u], acc_sh.at[dst_v.at[u]],
                                     s_sca.at[u], add=True)

            wait_scatter(0)
            wait_scatter(1)
            plsc.subcore_barrier()

            pltpu.sync_copy(
                acc_sh.at[pl.ds(s * ROWS_PER_SUB, ROWS_PER_SUB)],
                out_hbm.at[pl.ds(kk * NP + s * ROWS_PER_SUB, ROWS_PER_SUB)])

    return k(hc_flat, attr_flat, src, dst)


def _mlp_gn(h, agg_flat, w1t, b1, w2t, b2, epsl, gamma, beta, last):
    blk = NODE_BLK
    nblk = NP // blk
    gm = jnp.asarray(_GM)
    pm = jnp.asarray(_PM)

    def body(h_ref, a0_ref, a1_ref, a2_ref, a3_ref, w1_ref, b1_ref, w2_ref,
             b2_ref, gm_ref, pm_ref, gamma_ref, beta_ref, eps_ref, o_ref):
        h_blk = h_ref[...]
        a_refs = (a0_ref, a1_ref, a2_ref, a3_ref)
        agg_blk = jnp.concatenate(
            [a_refs[ci][:, :w] for ci, (c0, w) in enumerate(CHUNK_COLS)],
            axis=1)
        z = (1.0 + eps_ref[0, 0]) * h_blk + agg_blk
        z = jnp.maximum(
            jnp.dot(z, w1_ref[...], preferred_element_type=jnp.float32)
            + b1_ref[...], 0.0)
        z = jnp.dot(z, w2_ref[...], preferred_element_type=jnp.float32) \
            + b2_ref[...]
        mean = jnp.dot(z, gm_ref[...], preferred_element_type=jnp.float32)
        zc = z - jnp.dot(mean, pm_ref[...], preferred_element_type=jnp.float32)
        var = jnp.dot(zc * zc, gm_ref[...], preferred_element_type=jnp.float32)
        rstd = lax.rsqrt(var + 1e-5)
        zn = zc * jnp.dot(rstd, pm_ref[...], preferred_element_type=jnp.float32)
        out = zn * gamma_ref[...] + beta_ref[...]
        if not last:
            out = jnp.maximum(out, 0.0)
        o_ref[...] = out

    def chunk_spec(kc):
        return pl.BlockSpec((blk, CW), lambda i, kc=kc: (kc * nblk + i, 0))

    full = lambda shape: pl.BlockSpec(shape, lambda i: tuple(0 for _ in shape))
    in_specs = [
        pl.BlockSpec((blk, HIDDEN), lambda i: (i, 0)),
        chunk_spec(0), chunk_spec(1), chunk_spec(2), chunk_spec(3),
        full((HIDDEN, HIDDEN)),
        full((1, HIDDEN)),
        full((HIDDEN, HIDDEN)),
        full((1, HIDDEN)),
        full((HIDDEN, 10)),
        full((10, HIDDEN)),
        full((1, HIDDEN)),
        full((1, HIDDEN)),
        full((1, 1)),
    ]
    return pl.pallas_call(
        body,
        grid=(nblk,),
        in_specs=in_specs,
        out_specs=pl.BlockSpec((blk, HIDDEN), lambda i: (i, 0)),
        out_shape=jax.ShapeDtypeStruct((N_NODES, HIDDEN), jnp.float32),
    )(h, agg_flat, agg_flat, agg_flat, agg_flat, w1t, b1.reshape(1, HIDDEN),
      w2t, b2.reshape(1, HIDDEN), gm, pm, gamma.reshape(1, HIDDEN),
      beta.reshape(1, HIDDEN), epsl.reshape(1, 1))


def kernel(x, edge_index, edge_attr, W1, b1, W2, b2, eps, gamma, beta):
    src = edge_index[0]
    dst = edge_index[1]
    ac = _split_chunks_wide(edge_attr, EDGE_BLK, N_EDGES)
    hc = _split_chunks(x, NODE_BLK, NP)
    h = x
    for l in range(2):
        agg = _sc_agg(hc, ac, src, dst)
        h = _mlp_gn(h, agg, W1[l].T, b1[l], W2[l].T, b2[l], eps[l],
                    gamma[l], beta[l], last=(l == 1))
        if l == 0:
            hc = _split_chunks(h, NODE_BLK, NP)
    return h


# speedup vs baseline: 1.5306x; 1.1092x over previous
import functools

import jax
import jax.numpy as jnp
import numpy as np
from jax import lax
from jax.experimental import pallas as pl
from jax.experimental.pallas import tpu as pltpu
from jax.experimental.pallas import tpu_sc as plsc

N_NODES = 100000
N_EDGES = 1600000
HIDDEN = 50
NCHUNK = 4
CW = 16
CHUNK_COLS = ((0, 13), (13, 13), (26, 13), (39, 11))

N_SUB = 16
BLK_E = 400
EDGES_PER_SUB = N_EDGES // N_SUB
N_BLK = EDGES_PER_SUB // BLK_E
NP = 100096
ROWS_PER_SUB = NP // N_SUB
ZROWS = 136

NODE_BLK = 4352
EDGE_BLK = 6400

_SEL = np.zeros((NCHUNK, HIDDEN, CW), np.float32)
for _k, (_c0, _w) in enumerate(CHUNK_COLS):
    _SEL[_k, _c0 + np.arange(_w), np.arange(_w)] = 1.0

_g_of_c = np.arange(HIDDEN) // 5
_GM = np.zeros((HIDDEN, 10), np.float32)
_GM[np.arange(HIDDEN), _g_of_c] = 0.2
_PM = (_g_of_c[None, :] == np.arange(10)[:, None]).astype(np.float32)


def _split_chunks(x, blk, rows_out):
    n = x.shape[0]
    nblk = rows_out // blk

    def body(x_ref, s_ref, o_ref):
        o_ref[...] = jnp.dot(x_ref[...], s_ref[0],
                             preferred_element_type=jnp.float32)

    return pl.pallas_call(
        body,
        grid=(nblk, NCHUNK),
        in_specs=[
            pl.BlockSpec((blk, HIDDEN), lambda i, k: (i, 0)),
            pl.BlockSpec((1, HIDDEN, CW), lambda i, k: (k, 0, 0)),
        ],
        out_specs=pl.BlockSpec((blk, CW), lambda i, k: (k * nblk + i, 0)),
        out_shape=jax.ShapeDtypeStruct((NCHUNK * rows_out, CW), jnp.float32),
    )(x, jnp.asarray(_SEL))


def _split_chunks_wide(x, blk, rows_out, row_base=0):
    n = x.shape[0]
    nblk = rows_out // blk
    wblk = blk * CW // 128
    base_blk = row_base // blk

    def body(x_ref, s_ref, o_ref):
        for p in range(8):
            o_ref[:, CW * p:CW * (p + 1)] = jnp.dot(
                x_ref[:, p, :], s_ref[0], preferred_element_type=jnp.float32)

    return pl.pallas_call(
        body,
        grid=(nblk, NCHUNK),
        in_specs=[
            pl.BlockSpec((wblk, 8, HIDDEN), lambda i, k: (base_blk + i, 0, 0)),
            pl.BlockSpec((1, HIDDEN, CW), lambda i, k: (k, 0, 0)),
        ],
        out_specs=pl.BlockSpec((wblk, 128), lambda i, k: (k * nblk + i, 0)),
        out_shape=jax.ShapeDtypeStruct((NCHUNK * rows_out * CW // 128, 128),
                                       jnp.float32),
    )(x.reshape(n // 8, 8, HIDDEN), jnp.asarray(_SEL))


def _sc_agg(hc_flat, attr_flat, src, dst, acc_init, n_edges, e_base):
    mesh = plsc.VectorSubcoreMesh(core_axis_name="c", subcore_axis_name="s")
    edges_per_sub = n_edges // N_SUB
    n_blk = edges_per_sub // BLK_E

    @functools.partial(
        pl.kernel,
        out_type=jax.ShapeDtypeStruct((NCHUNK * NP, CW), jnp.float32),
        mesh=mesh,
        scratch_types=[
            pltpu.VMEM_SHARED((NP, CW), jnp.float32),
            pltpu.VMEM((2, BLK_E), jnp.int32),
            pltpu.VMEM((2, BLK_E), jnp.int32),
            pltpu.VMEM((2, BLK_E, CW), jnp.float32),
            pltpu.VMEM((2, BLK_E * CW // 128, 128), jnp.float32),
            pltpu.SemaphoreType.DMA((2,)),
            pltpu.SemaphoreType.DMA((2,)),
            pltpu.SemaphoreType.DMA((2,)),
            pltpu.SemaphoreType.DMA((2,)),
            pltpu.SemaphoreType.DMA((2,)),
        ],
        compiler_params=pltpu.CompilerParams(use_tc_tiling_on_sc=False),
    )
    def k(hc_hbm, attr_hbm, src_hbm, dst_hbm, init_hbm, out_hbm,
          acc_sh, src_v, dst_v, hrow_v, attr_v,
          s_src, s_dst, s_att, s_gat, s_sca):
        c = lax.axis_index("c")
        s = lax.axis_index("s")

        wblk = BLK_E * CW // 128

        def issue_in(jb, u, kk):
            e0 = e_base + s * edges_per_sub + jb * BLK_E
            w0 = (kk * n_edges + e0 - e_base) * CW // 128
            pltpu.async_copy(src_hbm.at[pl.ds(e0, BLK_E)], src_v.at[u],
                             s_src.at[u])
            pltpu.async_copy(dst_hbm.at[pl.ds(e0, BLK_E)], dst_v.at[u],
                             s_dst.at[u])
            pltpu.async_copy(attr_hbm.at[pl.ds(w0, wblk)],
                             attr_v.at[u], s_att.at[u])

        def wait_in(u):
            pltpu.make_async_copy(src_hbm.at[pl.ds(0, BLK_E)], src_v.at[u],
                                  s_src.at[u]).wait()
            pltpu.make_async_copy(dst_hbm.at[pl.ds(0, BLK_E)], dst_v.at[u],
                                  s_dst.at[u]).wait()
            pltpu.make_async_copy(attr_hbm.at[pl.ds(0, wblk)], attr_v.at[u],
                                  s_att.at[u]).wait()

        def wait_scatter(u):
            pltpu.make_async_copy(hrow_v.at[u], acc_sh.at[dst_v.at[u]],
                                  s_sca.at[u]).wait()

        for chunk_i in range(NCHUNK // 2):
            kk = 2 * c + chunk_i

            pltpu.sync_copy(
                init_hbm.at[pl.ds(kk * NP + s * ROWS_PER_SUB, ROWS_PER_SUB)],
                acc_sh.at[pl.ds(s * ROWS_PER_SUB, ROWS_PER_SUB)])

            plsc.subcore_barrier()

            off = kk * NP
            issue_in(0, 0, kk)

            def sub_iter(jb, u, kk):
                un = 1 - u
                @pl.when(jb + 1 < n_blk)
                def _():
                    @pl.when(jb >= 1)
                    def _():
                        wait_scatter(un)

                    issue_in(jb + 1, un, kk)

                wait_in(u)

                @pl.loop(0, BLK_E, step=16, unroll=8)
                def _(i):
                    src_v[u, pl.ds(i, 16)] = src_v[u, pl.ds(i, 16)] + off

                pltpu.async_copy(hc_hbm.at[src_v.at[u]], hrow_v.at[u],
                                 s_gat.at[u]).wait()

                @pl.loop(0, wblk, unroll=2)
                def _(q):
                    for r in range(8):
                        hrow_v[u, 8 * q + r] = jnp.maximum(
                            hrow_v[u, 8 * q + r]
                            + attr_v[u, q, 16 * r:16 * (r + 1)], 0.0)

                pltpu.async_copy(hrow_v.at[u], acc_sh.at[dst_v.at[u]],
                                 s_sca.at[u], add=True)

            @pl.loop(0, n_blk - (n_blk % 2), step=2)
            def _(j):
                for u in range(2):
                    sub_iter(j + u, u, kk)

            if n_blk % 2:
                sub_iter(n_blk - 1, 0, kk)
                wait_scatter(1)
                wait_scatter(0)
            else:
                wait_scatter(0)
                wait_scatter(1)
            plsc.subcore_barrier()

            pltpu.sync_copy(
                acc_sh.at[pl.ds(s * ROWS_PER_SUB, ROWS_PER_SUB)],
                out_hbm.at[pl.ds(kk * NP + s * ROWS_PER_SUB, ROWS_PER_SUB)])

    return k(hc_flat, attr_flat, src, dst, acc_init)


def _mlp_gn(h, agg_flat, w1t, b1, w2t, b2, epsl, gamma, beta, last):
    blk = NODE_BLK
    nblk = NP // blk
    gm = jnp.asarray(_GM)
    pm = jnp.asarray(_PM)

    def body(h_ref, a0_ref, a1_ref, a2_ref, a3_ref, w1_ref, b1_ref, w2_ref,
             b2_ref, gm_ref, pm_ref, gamma_ref, beta_ref, eps_ref, o_ref):
        h_blk = h_ref[...]
        a_refs = (a0_ref, a1_ref, a2_ref, a3_ref)
        agg_blk = jnp.concatenate(
            [a_refs[ci][:, :w] for ci, (c0, w) in enumerate(CHUNK_COLS)],
            axis=1)
        z = (1.0 + eps_ref[0, 0]) * h_blk + agg_blk
        z = jnp.maximum(
            jnp.dot(z, w1_ref[...], preferred_element_type=jnp.float32)
            + b1_ref[...], 0.0)
        z = jnp.dot(z, w2_ref[...], preferred_element_type=jnp.float32) \
            + b2_ref[...]
        mean = jnp.dot(z, gm_ref[...], preferred_element_type=jnp.float32)
        zc = z - jnp.dot(mean, pm_ref[...], preferred_element_type=jnp.float32)
        var = jnp.dot(zc * zc, gm_ref[...], preferred_element_type=jnp.float32)
        rstd = lax.rsqrt(var + 1e-5)
        zn = zc * jnp.dot(rstd, pm_ref[...], preferred_element_type=jnp.float32)
        out = zn * gamma_ref[...] + beta_ref[...]
        if not last:
            out = jnp.maximum(out, 0.0)
        o_ref[...] = out

    def chunk_spec(kc):
        return pl.BlockSpec((blk, CW), lambda i, kc=kc: (kc * nblk + i, 0))

    full = lambda shape: pl.BlockSpec(shape, lambda i: tuple(0 for _ in shape))
    in_specs = [
        pl.BlockSpec((blk, HIDDEN), lambda i: (i, 0)),
        chunk_spec(0), chunk_spec(1), chunk_spec(2), chunk_spec(3),
        full((HIDDEN, HIDDEN)),
        full((1, HIDDEN)),
        full((HIDDEN, HIDDEN)),
        full((1, HIDDEN)),
        full((HIDDEN, 10)),
        full((10, HIDDEN)),
        full((1, HIDDEN)),
        full((1, HIDDEN)),
        full((1, 1)),
    ]
    return pl.pallas_call(
        body,
        grid=(nblk,),
        in_specs=in_specs,
        out_specs=pl.BlockSpec((blk, HIDDEN), lambda i: (i, 0)),
        out_shape=jax.ShapeDtypeStruct((N_NODES, HIDDEN), jnp.float32),
    )(h, agg_flat, agg_flat, agg_flat, agg_flat, w1t, b1.reshape(1, HIDDEN),
      w2t, b2.reshape(1, HIDDEN), gm, pm, gamma.reshape(1, HIDDEN),
      beta.reshape(1, HIDDEN), epsl.reshape(1, 1))


def kernel(x, edge_index, edge_attr, W1, b1, W2, b2, eps, gamma, beta):
    src = edge_index[0]
    dst = edge_index[1]
    eh = N_EDGES // 2
    ac0 = _split_chunks_wide(edge_attr, EDGE_BLK, eh, row_base=0)
    hc = _split_chunks(x, NODE_BLK, NP)
    zeros = jnp.zeros((NCHUNK * NP, CW), jnp.float32)
    ac1 = _split_chunks_wide(edge_attr, EDGE_BLK, eh, row_base=eh)
    h = x
    for l in range(2):
        a0 = _sc_agg(hc, ac0, src, dst, zeros, eh, 0)
        agg = _sc_agg(hc, ac1, src, dst, a0, eh, eh)
        h = _mlp_gn(h, agg, W1[l].T, b1[l], W2[l].T, b2[l], eps[l],
                    gamma[l], beta[l], last=(l == 1))
        if l == 0:
            hc = _split_chunks(h, NODE_BLK, NP)
    return h
